# baseline (speedup 1.0000x reference)
import functools

import jax
import jax.numpy as jnp
from jax import lax
from jax.experimental import pallas as pl
from jax.experimental.pallas import tpu as pltpu
from jax.experimental.pallas import tpu_sc as plsc

WORD_SIZE = 100000
EMBED = 64
BATCH = 16384

NUM_CORES = 2
NUM_SUBCORES = 16
NUM_WORKERS = NUM_CORES * NUM_SUBCORES
FEATS_PER_W = EMBED // NUM_WORKERS
LANES = 16

_SEGS = [(0, 33280), (33280, 33280), (66560, 33408)]
TAIL_OFF = 99968
TAIL = WORD_SIZE - TAIL_OFF
SEG = 33408

_mesh = plsc.VectorSubcoreMesh(core_axis_name="c", subcore_axis_name="s")


@functools.partial(
    pl.kernel,
    mesh=_mesh,
    out_type=jax.ShapeDtypeStruct((EMBED, BATCH), jnp.float32),
    scratch_types=[
        pltpu.VMEM((SEG,), jnp.float32),
        pltpu.VMEM((SEG,), jnp.float32),
        pltpu.VMEM((FEATS_PER_W * TAIL,), jnp.float32),
        pltpu.VMEM((BATCH,), jnp.int32),
        pltpu.VMEM((BATCH,), jnp.float32),
        pltpu.SemaphoreType.DMA,
        pltpu.SemaphoreType.DMA,
        pltpu.SemaphoreType.DMA,
        pltpu.SemaphoreType.DMA,
    ],
    compiler_params=pltpu.CompilerParams(
        use_tc_tiling_on_sc=True, needs_layout_passes=False
    ),
)
def _embed_gather(
    tab_t_hbm,
    idx_hbm,
    tail_hbm,
    out_t_hbm,
    row_a,
    row_b,
    tail_v,
    idx_v,
    out_v,
    sem_i,
    sem_a,
    sem_b,
    sem_o,
):
    wid = lax.axis_index("s") * NUM_CORES + lax.axis_index("c")
    f0 = wid * FEATS_PER_W
    bufs = [row_a, row_b]
    sems = [sem_a, sem_b]
    passes = [(f, s) for f in range(FEATS_PER_W) for s in range(len(_SEGS))]
    iota = lax.iota(jnp.int32, LANES)

    idx_cp = pltpu.async_copy(idx_hbm, idx_v, sem_i)
    tail_cps = [
        pltpu.async_copy(
            tail_hbm.at[pl.ds((f0 + f) * TAIL, TAIL)],
            tail_v.at[pl.ds(f * TAIL, TAIL)],
            sem_i,
        )
        for f in range(FEATS_PER_W)
    ]
    off0, sz0 = _SEGS[0]
    pending = pltpu.async_copy(
        tab_t_hbm.at[f0, pl.ds(off0, sz0)], bufs[0].at[pl.ds(0, sz0)], sems[0]
    )
    idx_cp.wait()
    for cp in tail_cps:
        cp.wait()

    out_cp = None
    for p, (f, s) in enumerate(passes):
        buf = bufs[p % 2]
        pending.wait()
        if p + 1 < len(passes):
            nf, ns = passes[p + 1]
            noff, nsz = _SEGS[ns]
            pending = pltpu.async_copy(
                tab_t_hbm.at[f0 + nf, pl.ds(noff, nsz)],
                bufs[(p + 1) % 2].at[pl.ds(0, nsz)],
                sems[(p + 1) % 2],
            )
        off, sz = _SEGS[s]
        if s == 0 and out_cp is not None:
            out_cp.wait()

        if s == 0:

            @plsc.parallel_loop(0, BATCH, step=LANES)
            def _body0(j):
                iv = idx_v[pl.ds(j, LANES)]
                m = plsc.bitcast(iv, jnp.uint32) < jnp.uint32(sz)
                out_v[pl.ds(j, LANES)] = plsc.load_gather(buf, [iv], mask=m)

        else:

            @plsc.parallel_loop(0, BATCH, step=LANES)
            def _body(j):
                iv = idx_v[pl.ds(j, LANES)] - off
                m = plsc.bitcast(iv, jnp.uint32) < jnp.uint32(sz)
                g = plsc.load_gather(buf, [iv], mask=m)
                plsc.store_scatter(out_v, [j + iota], g, mask=m)

        if s == len(_SEGS) - 1:

            @plsc.parallel_loop(0, BATCH, step=LANES)
            def _tail(j):
                iv = idx_v[pl.ds(j, LANES)] - TAIL_OFF
                m = plsc.bitcast(iv, jnp.uint32) < jnp.uint32(TAIL)
                g = plsc.load_gather(tail_v, [iv + (f * TAIL)], mask=m)
                plsc.store_scatter(out_v, [j + iota], g, mask=m)

            out_cp = pltpu.async_copy(out_v, out_t_hbm.at[f0 + f], sem_o)
    out_cp.wait()


def kernel(inputs, table):
    idx = inputs.reshape(BATCH).astype(jnp.int32)
    tab_t = table.T
    tail = tab_t[:, TAIL_OFF:].reshape(-1)
    out_t = _embed_gather(tab_t, idx, tail)
    return out_t.T


# speedup vs baseline: 1.5202x; 1.5202x over previous
import functools

import jax
import jax.numpy as jnp
from jax import lax
from jax.experimental import pallas as pl
from jax.experimental.pallas import tpu as pltpu
from jax.experimental.pallas import tpu_sc as plsc

WORD_SIZE = 100000
EMBED = 64
BATCH = 16384

NUM_CORES = 2
NUM_SUBCORES = 16
NUM_WORKERS = NUM_CORES * NUM_SUBCORES
FEATS_PER_W = EMBED // NUM_WORKERS
LANES = 16
OUT_CHUNK = 4096
N_CHUNKS = BATCH // OUT_CHUNK

_mesh = plsc.VectorSubcoreMesh(core_axis_name="c", subcore_axis_name="s")


@functools.partial(
    pl.kernel,
    mesh=_mesh,
    out_type=jax.ShapeDtypeStruct((EMBED, BATCH), jnp.float32),
    scratch_types=[
        pltpu.VMEM((WORD_SIZE,), jnp.float32),
        pltpu.VMEM((BATCH,), jnp.int32),
        pltpu.VMEM((OUT_CHUNK,), jnp.float32),
        pltpu.VMEM((OUT_CHUNK,), jnp.float32),
        pltpu.SemaphoreType.DMA,
        pltpu.SemaphoreType.DMA,
        pltpu.SemaphoreType.DMA,
        pltpu.SemaphoreType.DMA,
    ],
    compiler_params=pltpu.CompilerParams(
        use_tc_tiling_on_sc=True, needs_layout_passes=False
    ),
)
def _embed_gather(
    tab_t_hbm, idx_hbm, out_t_hbm, row_v, idx_v, out_a, out_b, sem_i, sem_r, sem_a, sem_b
):
    wid = lax.axis_index("s") * NUM_CORES + lax.axis_index("c")
    f0 = wid * FEATS_PER_W
    out_bufs = [out_a, out_b]
    out_sems = [sem_a, sem_b]

    idx_cp = pltpu.async_copy(idx_hbm, idx_v, sem_i)
    row_cp = pltpu.async_copy(tab_t_hbm.at[f0], row_v, sem_r)
    idx_cp.wait()

    flushes = {}
    for f in range(FEATS_PER_W):
        row_cp.wait()
        for c in range(N_CHUNKS):
            buf = out_bufs[c % 2]
            base = c * OUT_CHUNK
            prior = flushes.pop(c % 2, None)
            if prior is not None:
                prior.wait()

            @plsc.parallel_loop(0, OUT_CHUNK, step=LANES)
            def _body(j):
                iv = idx_v[pl.ds(base + j, LANES)]
                buf[pl.ds(j, LANES)] = plsc.load_gather(row_v, [iv])

            if f + 1 < FEATS_PER_W and c == N_CHUNKS - 1:
                row_cp = pltpu.async_copy(tab_t_hbm.at[f0 + f + 1], row_v, sem_r)
            flushes[c % 2] = pltpu.async_copy(
                buf, out_t_hbm.at[f0 + f, pl.ds(base, OUT_CHUNK)], out_sems[c % 2]
            )
    for cp in flushes.values():
        cp.wait()


def kernel(inputs, table):
    idx = inputs.reshape(BATCH).astype(jnp.int32)
    out_t = _embed_gather(table.T, idx)
    return out_t.T


# speedup vs baseline: 1.9007x; 1.2503x over previous
import functools

import jax
import jax.numpy as jnp
from jax import lax
from jax.experimental import pallas as pl
from jax.experimental.pallas import tpu as pltpu
from jax.experimental.pallas import tpu_sc as plsc

WORD_SIZE = 100000
EMBED = 64
BATCH = 16384

NUM_CORES = 2
NUM_SUBCORES = 16
NUM_WORKERS = NUM_CORES * NUM_SUBCORES
FEATS_PER_W = EMBED // NUM_WORKERS
LANES = 16
OUT_CHUNK = 4096
N_CHUNKS = BATCH // OUT_CHUNK

_mesh = plsc.VectorSubcoreMesh(core_axis_name="c", subcore_axis_name="s")


@functools.partial(
    pl.kernel,
    mesh=_mesh,
    out_type=jax.ShapeDtypeStruct((EMBED, BATCH), jnp.float32),
    scratch_types=[
        pltpu.VMEM((WORD_SIZE,), jnp.float32),
        pltpu.VMEM((BATCH,), jnp.int32),
        pltpu.VMEM((OUT_CHUNK,), jnp.float32),
        pltpu.VMEM((OUT_CHUNK,), jnp.float32),
        pltpu.SemaphoreType.DMA,
        pltpu.SemaphoreType.DMA,
        pltpu.SemaphoreType.DMA,
        pltpu.SemaphoreType.DMA,
    ],
    compiler_params=pltpu.CompilerParams(
        use_tc_tiling_on_sc=True, needs_layout_passes=False
    ),
)
def _embed_gather(
    tab_t_hbm, idx_hbm, out_t_hbm, row_v, idx_v, out_a, out_b, sem_i, sem_r, sem_a, sem_b
):
    wid = lax.axis_index("s") * NUM_CORES + lax.axis_index("c")
    f0 = wid * FEATS_PER_W
    out_bufs = [out_a, out_b]
    out_sems = [sem_a, sem_b]

    idx_cp = pltpu.async_copy(idx_hbm, idx_v, sem_i)
    row_cp = pltpu.async_copy(tab_t_hbm.at[f0], row_v, sem_r)
    idx_cp.wait()

    flushes = {}
    for f in range(FEATS_PER_W):
        row_cp.wait()
        for c in range(N_CHUNKS):
            buf = out_bufs[c % 2]
            base = c * OUT_CHUNK
            prior = flushes.pop(c % 2, None)
            if prior is not None:
                prior.wait()

            @plsc.parallel_loop(0, OUT_CHUNK, step=LANES, unroll=8)
            def _body(j):
                iv = idx_v[pl.ds(base + j, LANES)]
                buf[pl.ds(j, LANES)] = plsc.load_gather(row_v, [iv])

            if f + 1 < FEATS_PER_W and c == N_CHUNKS - 1:
                row_cp = pltpu.async_copy(tab_t_hbm.at[f0 + f + 1], row_v, sem_r)
            flushes[c % 2] = pltpu.async_copy(
                buf, out_t_hbm.at[f0 + f, pl.ds(base, OUT_CHUNK)], out_sems[c % 2]
            )
    for cp in flushes.values():
        cp.wait()


def kernel(inputs, table):
    idx = inputs.reshape(BATCH).astype(jnp.int32)
    out_t = _embed_gather(table.T, idx)
    return out_t.T
